# baseline (speedup 1.0000x reference)
import dataclasses

import jax
import jax.numpy as jnp
from jax import lax
from jax.experimental import pallas as pl
from jax.experimental.pallas import tpu as pltpu
from jax.experimental.pallas import tpu_sc as plsc

_B = 16384
_S = 168
_BR = 32
_TAB_PAD = 9216

_YEAR0 = 2015

_COL_STARTS = tuple(range(0, _S - 16, 16)) + (_S - 16,)


def _sc_lookup(tab_pad, slope16, weeks, weekdays, hours, years):
    mesh = plsc.VectorSubcoreMesh(core_axis_name="c", subcore_axis_name="s")
    cp = pltpu.CompilerParams()
    if "needs_layout_passes" in pltpu.CompilerParams.__dataclass_fields__:
        cp = dataclasses.replace(cp, needs_layout_passes=False)

    @pl.kernel(
        compiler_params=cp,
        out_type=jax.ShapeDtypeStruct((_B, _S), jnp.float32),
        mesh=mesh,
        scratch_types=[
            pltpu.VMEM((_TAB_PAD,), jnp.float32),
            pltpu.VMEM((16,), jnp.float32),
        ],
    )
    def k(tab_hbm, slope_hbm, w_hbm, d_hbm, h_hbm, y_hbm, o_hbm, tab_v, slope_v):
        pltpu.sync_copy(tab_hbm, tab_v)
        pltpu.sync_copy(slope_hbm, slope_v)

        def body(w_ref, d_ref, h_ref, y_ref, o_ref):
            sv = slope_v[...]

            @pl.loop(0, _BR)
            def _(r):
                for c in _COL_STARTS:
                    s = pl.ds(c, 16)
                    w = w_ref[r, s]
                    d = d_ref[r, s]
                    h = h_ref[r, s]
                    y = y_ref[r, s]
                    idx = w * 168 + d * 24 + h
                    base = plsc.load_gather(tab_v, [idx])
                    yf = (y - _YEAR0).astype(jnp.float32)
                    o_ref[r, s] = base + sv * yf

        spec = pl.BlockSpec((_BR, _S), lambda i: (i, 0))
        pltpu.emit_pipeline(
            body,
            grid=(_B // _BR,),
            in_specs=[spec, spec, spec, spec],
            out_specs=[spec],
            core_axis_name=("c", "s"),
            dimension_semantics=(pltpu.PARALLEL,),
        )(w_hbm, d_hbm, h_hbm, y_hbm, o_hbm)

    return k(tab_pad, slope16, weeks, weekdays, hours, years)


@jax.jit
def kernel(loaddata, weeks, years, weekdays, hours, seasonal_delta,
           cosmic_slope, cosmic_intersection):
    del loaddata
    flat = seasonal_delta.reshape(-1) + cosmic_intersection
    tab_pad = jnp.zeros((_TAB_PAD,), jnp.float32)
    tab_pad = lax.dynamic_update_slice(tab_pad, flat, (192,))
    slope16 = jnp.full((16,), cosmic_slope, jnp.float32)

    out = _sc_lookup(tab_pad, slope16, weeks, weekdays, hours, years)
    return out[..., None]


# speedup vs baseline: 1.3728x; 1.3728x over previous
import dataclasses

import jax
import jax.numpy as jnp
from jax import lax
from jax.experimental import pallas as pl
from jax.experimental.pallas import tpu as pltpu
from jax.experimental.pallas import tpu_sc as plsc

_B = 16384
_S = 168
_BR = 32
_TAB_PAD = 9216

_YEAR0 = 2015

_COL_STARTS = tuple(range(0, _S - 16, 16)) + (_S - 16,)


def _sc_lookup(tab_pad, slope16, weeks, weekdays, hours, years):
    mesh = plsc.VectorSubcoreMesh(core_axis_name="c", subcore_axis_name="s")
    cp = pltpu.CompilerParams()
    if "needs_layout_passes" in pltpu.CompilerParams.__dataclass_fields__:
        cp = dataclasses.replace(cp, needs_layout_passes=False)

    @pl.kernel(
        compiler_params=cp,
        out_type=jax.ShapeDtypeStruct((_B, _S), jnp.float32),
        mesh=mesh,
        scratch_types=[
            pltpu.VMEM((_TAB_PAD,), jnp.float32),
            pltpu.VMEM((16,), jnp.float32),
        ],
    )
    def k(tab_hbm, slope_hbm, w_hbm, d_hbm, h_hbm, y_hbm, o_hbm, tab_v, slope_v):
        pltpu.sync_copy(tab_hbm, tab_v)
        pltpu.sync_copy(slope_hbm, slope_v)

        def body(w_ref, d_ref, h_ref, y_ref, o_ref):
            sv = slope_v[...]

            @plsc.parallel_loop(0, _BR, 1, unroll=2)
            def _(r):
                for c in _COL_STARTS:
                    s = pl.ds(c, 16)
                    w = w_ref[r, s]
                    d = d_ref[r, s]
                    h = h_ref[r, s]
                    y = y_ref[r, s]
                    idx = w * 168 + d * 24 + h
                    base = plsc.load_gather(tab_v, [idx])
                    yf = (y - _YEAR0).astype(jnp.float32)
                    o_ref[r, s] = base + sv * yf

        spec = pl.BlockSpec((_BR, _S), lambda i: (i, 0))
        pltpu.emit_pipeline(
            body,
            grid=(_B // _BR,),
            in_specs=[spec, spec, spec, spec],
            out_specs=[spec],
            core_axis_name=("c", "s"),
            dimension_semantics=(pltpu.PARALLEL,),
        )(w_hbm, d_hbm, h_hbm, y_hbm, o_hbm)

    return k(tab_pad, slope16, weeks, weekdays, hours, years)


@jax.jit
def kernel(loaddata, weeks, years, weekdays, hours, seasonal_delta,
           cosmic_slope, cosmic_intersection):
    del loaddata
    flat = seasonal_delta.reshape(-1) + cosmic_intersection
    tab_pad = jnp.zeros((_TAB_PAD,), jnp.float32)
    tab_pad = lax.dynamic_update_slice(tab_pad, flat, (192,))
    slope16 = jnp.full((16,), cosmic_slope, jnp.float32)

    out = _sc_lookup(tab_pad, slope16, weeks, weekdays, hours, years)
    return out[..., None]
